# baseline (speedup 1.0000x reference)
import functools

import numpy as np
import jax
import jax.numpy as jnp
from jax import lax
from jax.experimental import pallas as pl
from jax.experimental.pallas import tpu as pltpu
from jax.experimental.pallas import tpu_sc as plsc

_VOCAB = 1000
_VPAD = 1024
_BLOCK = 256
_DEPTH = 4
_H = 4096
_B = 16

_NW = 32
_POS = _B * _BLOCK
_PPW = _POS // _NW
_G = 16
_NG = _PPW // _G
_NFULL = _VOCAB // 16

_TBITS = 11
_TSHIFT = 23 - _TBITS
_TBASE = 126 << _TBITS
_TSIZE = 8 << _TBITS


def _make_log_table() -> np.ndarray:
    import ml_dtypes

    j = np.arange(_TSIZE)
    e = (j >> _TBITS).astype(np.float64) - 1.0
    m = 1.0 + ((j & ((1 << _TBITS) - 1)).astype(np.float64) + 0.5) / (1 << _TBITS)
    vals = ((e + np.log2(m) + 1.0) * np.log(2.0)).astype(np.float32)
    return vals.astype(ml_dtypes.bfloat16).astype(np.float32)


_LOG_TAB = _make_log_table()


def _tix(c):
    bits = lax.bitcast_convert_type(c, jnp.int32)
    return jnp.clip((bits >> _TSHIFT) - _TBASE, 0, _TSIZE - 1)


def _sc_body(idx_hbm, tab3_hbm, wp_hbm, ltab_hbm, out_hbm,
             ibuf, rbuf, grows0, grows1, obuf, cbuf, tbuf, wbuf,
             sem0, sem1, semo):
    cid = lax.axis_index("c")
    sid = lax.axis_index("s")
    w = sid * 2 + cid
    bb = w // 2
    half = w % 2
    t0 = half * (_BLOCK // 2)

    pltpu.sync_copy(wp_hbm, wbuf)
    pltpu.sync_copy(ltab_hbm, tbuf)

    @pl.when(half == 0)
    def _():
        pltpu.sync_copy(idx_hbm.at[pl.ds(bb * _BLOCK, 128)], ibuf.at[pl.ds(8, 128)])

    @pl.when(half == 1)
    def _():
        pltpu.sync_copy(idx_hbm.at[pl.ds(bb * _BLOCK + 120, 136)], ibuf.at[pl.ds(0, 136)])

    iota = lax.iota(jnp.int32, 16)
    zero16 = jnp.zeros((16,), jnp.int32)

    @pl.when(half == 0)
    def _():
        plsc.store_scatter(ibuf, [iota], zero16, mask=iota < 8)

    wvec = wbuf[pl.ds(0, 16)]
    W0 = wvec[0]
    W1 = wvec[1]
    W2 = wvec[2]
    W3 = wvec[3]
    bias = wvec[4]

    for g in range(_NG):
        o1 = g * _G
        tm1 = plsc.load_gather(ibuf, [iota + (o1 + 7)])
        tm2 = plsc.load_gather(ibuf, [iota + (o1 + 6)])
        tm3 = plsc.load_gather(ibuf, [iota + (o1 + 5)])
        h12 = tm1 + tm2 * 31
        rbuf[3 * g + 0, pl.ds(0, 16)] = tm1 & (_H - 1)
        rbuf[3 * g + 1, pl.ds(0, 16)] = h12 & (_H - 1)
        rbuf[3 * g + 2, pl.ds(0, 16)] = (h12 + tm3 * 961) & (_H - 1)

    def gather_descs(g, grows, sem):
        return [
            pltpu.make_async_copy(
                tab3_hbm.at[d + 1].at[rbuf.at[3 * g + d]],
                grows.at[pl.ds(d * _G, _G), :],
                sem,
            )
            for d in range(3)
        ]

    def fire_gather(g, grows, sem):
        for desc in gather_descs(g, grows, sem):
            desc.start()

    def drain_gather(g, grows, sem):
        for desc in gather_descs(g, grows, sem):
            desc.wait()

    fire_gather(0, grows0, sem0)
    fire_gather(1, grows1, sem1)

    pltpu.sync_copy(tab3_hbm.at[0, 0], obuf.at[0])
    w0v = jnp.full((16,), W0, jnp.float32)
    bv = jnp.full((16,), bias, jnp.float32)

    def cinit(jj):
        c0 = obuf[0, pl.ds(jj, 16)]
        lk = plsc.load_gather(tbuf, [_tix(c0)])
        cbuf[pl.ds(jj, 16)] = bv + w0v * lk

    @plsc.parallel_loop(0, _NFULL)
    def _(j):
        cinit(j * 16)

    cinit(_VOCAB - 16)

    def compute_group(g, grows):
        tg = t0 + g * _G

        @pl.when(g > 0)
        def _():
            pltpu.make_async_copy(
                obuf, out_hbm.at[bb, pl.ds(tg - _G, _G), :], semo
            ).wait()

        for p in range(_G):
            t = tg + p
            wb1 = jnp.full((16,), jnp.where(t >= 1, W1, 0.0), jnp.float32)
            wb2 = jnp.full((16,), jnp.where(t >= 2, W2, 0.0), jnp.float32)
            wb3 = jnp.full((16,), jnp.where(t >= 3, W3, 0.0), jnp.float32)

            def chunk(jj):
                c1 = grows[p, pl.ds(jj, 16)]
                c2 = grows[p + _G, pl.ds(jj, 16)]
                c3 = grows[p + 2 * _G, pl.ds(jj, 16)]
                t1 = plsc.load_gather(tbuf, [_tix(c1)])
                t2 = plsc.load_gather(tbuf, [_tix(c2)])
                t3 = plsc.load_gather(tbuf, [_tix(c3)])
                cb = cbuf[pl.ds(jj, 16)]
                obuf[p, pl.ds(jj, 16)] = cb + wb1 * t1 + wb2 * t2 + wb3 * t3

            @plsc.parallel_loop(0, _NFULL, unroll=2)
            def _(j):
                chunk(j * 16)

            chunk(_VOCAB - 16)

        pltpu.make_async_copy(
            obuf, out_hbm.at[bb, pl.ds(tg, _G), :], semo
        ).start()

    def pair(k, _):
        g0 = 2 * k
        g1 = 2 * k + 1

        drain_gather(g0, grows0, sem0)
        compute_group(g0, grows0)

        @pl.when(k < _NG // 2 - 1)
        def _():
            fire_gather(g0 + 2, grows0, sem0)

        drain_gather(g1, grows1, sem1)
        compute_group(g1, grows1)

        @pl.when(k < _NG // 2 - 1)
        def _():
            fire_gather(g1 + 2, grows1, sem1)

        return 0

    lax.fori_loop(0, _NG // 2, pair, 0)

    pltpu.make_async_copy(
        obuf, out_hbm.at[bb, pl.ds(t0 + _PPW - _G, _G), :], semo
    ).wait()


@functools.partial(
    pl.kernel,
    mesh=plsc.VectorSubcoreMesh(core_axis_name="c", subcore_axis_name="s"),
    out_type=jax.ShapeDtypeStruct((_B, _BLOCK, _VOCAB), jnp.float32),
    compiler_params=pltpu.CompilerParams(
        needs_layout_passes=False, use_tc_tiling_on_sc=False
    ),
    scratch_types=[
        pltpu.VMEM((144,), jnp.int32),
        pltpu.VMEM((3 * _NG, _G), jnp.int32),
        pltpu.VMEM((3 * _G, _VOCAB), jnp.float32),
        pltpu.VMEM((3 * _G, _VOCAB), jnp.float32),
        pltpu.VMEM((_G, _VOCAB), jnp.float32),
        pltpu.VMEM((_VOCAB,), jnp.float32),
        pltpu.VMEM((_TSIZE,), jnp.float32),
        pltpu.VMEM((16,), jnp.float32),
        pltpu.SemaphoreType.DMA,
        pltpu.SemaphoreType.DMA,
        pltpu.SemaphoreType.DMA,
    ],
)
def _sc_call(idx_hbm, tab_hbm, wp_hbm, ltab_hbm, out_hbm, *scratch):
    _sc_body(idx_hbm, tab_hbm, wp_hbm, ltab_hbm, out_hbm, *scratch)


def kernel(idx, counts_table, linear_w, linear_b):
    w_bf = linear_w[0].astype(jnp.bfloat16).astype(jnp.float32)
    wp = jnp.pad(jnp.concatenate([w_bf, linear_b]).astype(jnp.float32), (0, 11))
    return _sc_call(idx.reshape(-1), counts_table, wp, jnp.asarray(_LOG_TAB))


# speedup vs baseline: 1.1726x; 1.1726x over previous
import functools

import numpy as np
import jax
import jax.numpy as jnp
from jax import lax
from jax.experimental import pallas as pl
from jax.experimental.pallas import tpu as pltpu
from jax.experimental.pallas import tpu_sc as plsc

_VOCAB = 1000
_VPAD = 1024
_BLOCK = 256
_DEPTH = 4
_H = 4096
_B = 16

_NW = 32
_POS = _B * _BLOCK
_PPW = _POS // _NW
_G = 16
_NG = _PPW // _G
_NFULL = _VOCAB // 16

_TBITS = 11
_TSHIFT = 23 - _TBITS
_TBASE = 126 << _TBITS
_TSIZE = 8 << _TBITS


def _make_log_table() -> np.ndarray:
    import ml_dtypes

    j = np.arange(_TSIZE)
    e = (j >> _TBITS).astype(np.float64) - 1.0
    m = 1.0 + ((j & ((1 << _TBITS) - 1)).astype(np.float64) + 0.5) / (1 << _TBITS)
    vals = ((e + np.log2(m) + 1.0) * np.log(2.0)).astype(np.float32)
    return vals.astype(ml_dtypes.bfloat16).astype(np.float32)


_LOG_TAB = _make_log_table()


def _tix(c):
    bits = lax.bitcast_convert_type(c, jnp.int32)
    return jnp.clip((bits >> _TSHIFT) - _TBASE, 0, _TSIZE - 1)


def _sc_body(idx_hbm, tab1_hbm, tab23_hbm, row0_hbm, wp_hbm, ltab_hbm, out_hbm,
             ibuf, rbuf, grows0, grows1, obuf, cbuf, tbuf, wbuf,
             sem0, sem1, semo):
    cid = lax.axis_index("c")
    sid = lax.axis_index("s")
    w = sid * 2 + cid
    bb = w // 2
    half = w % 2
    t0 = half * (_BLOCK // 2)

    pltpu.sync_copy(wp_hbm, wbuf)
    pltpu.sync_copy(ltab_hbm, tbuf)

    @pl.when(half == 0)
    def _():
        pltpu.sync_copy(idx_hbm.at[pl.ds(bb * _BLOCK, 128)], ibuf.at[pl.ds(8, 128)])

    @pl.when(half == 1)
    def _():
        pltpu.sync_copy(idx_hbm.at[pl.ds(bb * _BLOCK + 120, 136)], ibuf.at[pl.ds(0, 136)])

    iota = lax.iota(jnp.int32, 16)
    zero16 = jnp.zeros((16,), jnp.int32)

    @pl.when(half == 0)
    def _():
        plsc.store_scatter(ibuf, [iota], zero16, mask=iota < 8)

    wvec = wbuf[pl.ds(0, 16)]
    W0 = wvec[0]
    W1 = wvec[1]
    W2 = wvec[2]
    W3 = wvec[3]
    bias = wvec[4]

    for g in range(_NG):
        o1 = g * _G
        tm1 = plsc.load_gather(ibuf, [iota + (o1 + 7)])
        tm2 = plsc.load_gather(ibuf, [iota + (o1 + 6)])
        tm3 = plsc.load_gather(ibuf, [iota + (o1 + 5)])
        h12 = tm1 + tm2 * 31
        rbuf[3 * g + 0, pl.ds(0, 16)] = tm1 & (_H - 1)
        rbuf[3 * g + 1, pl.ds(0, 16)] = h12 & (_H - 1)
        rbuf[3 * g + 2, pl.ds(0, 16)] = (h12 + tm3 * 961) & (_H - 1)

    def gather_descs(g, grows, sem):
        return [
            pltpu.make_async_copy(
                tab1_hbm.at[rbuf.at[3 * g + 0]],
                grows.at[pl.ds(0, _G), :],
                sem,
            ),
            pltpu.make_async_copy(
                tab23_hbm.at[0].at[rbuf.at[3 * g + 1]],
                grows.at[pl.ds(_G, _G), :],
                sem,
            ),
            pltpu.make_async_copy(
                tab23_hbm.at[1].at[rbuf.at[3 * g + 2]],
                grows.at[pl.ds(2 * _G, _G), :],
                sem,
            ),
        ]

    def fire_gather(g, grows, sem):
        for desc in gather_descs(g, grows, sem):
            desc.start()

    def drain_gather(g, grows, sem):
        for desc in gather_descs(g, grows, sem):
            desc.wait()

    fire_gather(0, grows0, sem0)
    fire_gather(1, grows1, sem1)

    pltpu.sync_copy(row0_hbm, obuf.at[0])
    w0v = jnp.full((16,), W0, jnp.float32)
    bv = jnp.full((16,), bias, jnp.float32)

    def cinit(jj):
        c0 = obuf[0, pl.ds(jj, 16)]
        lk = plsc.load_gather(tbuf, [_tix(c0)])
        cbuf[pl.ds(jj, 16)] = bv + w0v * lk

    @plsc.parallel_loop(0, _NFULL)
    def _(j):
        cinit(j * 16)

    cinit(_VOCAB - 16)

    def compute_group(g, grows):
        tg = t0 + g * _G

        @pl.when(g > 0)
        def _():
            pltpu.make_async_copy(
                obuf, out_hbm.at[bb, pl.ds(tg - _G, _G), :], semo
            ).wait()

        for p in range(_G):
            t = tg + p
            wb1 = jnp.full((16,), jnp.where(t >= 1, W1, 0.0), jnp.float32)
            wb2 = jnp.full((16,), jnp.where(t >= 2, W2, 0.0), jnp.float32)
            wb3 = jnp.full((16,), jnp.where(t >= 3, W3, 0.0), jnp.float32)

            def chunk(jj):
                c1 = grows[p, pl.ds(jj, 16)]
                c2 = grows[p + _G, pl.ds(jj, 16)]
                c3 = grows[p + 2 * _G, pl.ds(jj, 16)]
                t1 = plsc.load_gather(tbuf, [_tix(c1)])
                t2 = plsc.load_gather(tbuf, [_tix(c2)])
                t3 = plsc.load_gather(tbuf, [_tix(c3)])
                cb = cbuf[pl.ds(jj, 16)]
                obuf[p, pl.ds(jj, 16)] = cb + wb1 * t1 + wb2 * t2 + wb3 * t3

            @plsc.parallel_loop(0, _NFULL, unroll=2)
            def _(j):
                chunk(j * 16)

            chunk(_VOCAB - 16)

        pltpu.make_async_copy(
            obuf, out_hbm.at[bb, pl.ds(tg, _G), :], semo
        ).start()

    def pair(k, _):
        g0 = 2 * k
        g1 = 2 * k + 1

        drain_gather(g0, grows0, sem0)
        compute_group(g0, grows0)

        @pl.when(k < _NG // 2 - 1)
        def _():
            fire_gather(g0 + 2, grows0, sem0)

        drain_gather(g1, grows1, sem1)
        compute_group(g1, grows1)

        @pl.when(k < _NG // 2 - 1)
        def _():
            fire_gather(g1 + 2, grows1, sem1)

        return 0

    lax.fori_loop(0, _NG // 2, pair, 0)

    pltpu.make_async_copy(
        obuf, out_hbm.at[bb, pl.ds(t0 + _PPW - _G, _G), :], semo
    ).wait()


@functools.partial(
    pl.kernel,
    mesh=plsc.VectorSubcoreMesh(core_axis_name="c", subcore_axis_name="s"),
    out_type=jax.ShapeDtypeStruct((_B, _BLOCK, _VOCAB), jnp.float32),
    compiler_params=pltpu.CompilerParams(
        needs_layout_passes=False, use_tc_tiling_on_sc=False
    ),
    scratch_types=[
        pltpu.VMEM((144,), jnp.int32),
        pltpu.VMEM((3 * _NG, _G), jnp.int32),
        pltpu.VMEM((3 * _G, _VOCAB), jnp.float32),
        pltpu.VMEM((3 * _G, _VOCAB), jnp.float32),
        pltpu.VMEM((_G, _VOCAB), jnp.float32),
        pltpu.VMEM((_VOCAB,), jnp.float32),
        pltpu.VMEM((_TSIZE,), jnp.float32),
        pltpu.VMEM((16,), jnp.float32),
        pltpu.SemaphoreType.DMA,
        pltpu.SemaphoreType.DMA,
        pltpu.SemaphoreType.DMA,
    ],
)
def _sc_call(idx_hbm, tab1_hbm, tab23_hbm, row0_hbm, wp_hbm, ltab_hbm, out_hbm,
             *scratch):
    _sc_body(idx_hbm, tab1_hbm, tab23_hbm, row0_hbm, wp_hbm, ltab_hbm, out_hbm,
             *scratch)


def kernel(idx, counts_table, linear_w, linear_b):
    w_bf = linear_w[0].astype(jnp.bfloat16).astype(jnp.float32)
    wp = jnp.pad(jnp.concatenate([w_bf, linear_b]).astype(jnp.float32), (0, 11))
    return _sc_call(
        idx.reshape(-1),
        counts_table[1, :_VOCAB],
        counts_table[2:],
        counts_table[0, 0],
        wp,
        jnp.asarray(_LOG_TAB),
    )
